# baseline (speedup 1.0000x reference)
import jax
import jax.numpy as jnp
from jax.experimental import pallas as pl
from jax.experimental.pallas import tpu as pltpu
from jax.experimental.pallas import tpu_sc as plsc

_E = 8
_EL = 16
_D = 1024
_O = 1024
_B = 2048
_SCALE = 20.0
_TM = 128
_SW = 128
_SPLIT = 4
_SD = _D // _SPLIT
_SB = _B * _SPLIT


def _route_body(idx_ref, pos_ref, perm_ref, off_ref):
    idx = idx_ref[...]
    lane = jax.lax.broadcasted_iota(jnp.int32, (1, _EL), 1)
    onehot_b = (idx == lane).astype(jnp.bfloat16)
    r = jax.lax.broadcasted_iota(jnp.int32, (_B, 1), 0)
    c = jax.lax.broadcasted_iota(jnp.int32, (1, _B), 1)
    tril = (r >= c).astype(jnp.bfloat16)
    cum = jax.lax.dot_general(
        tril, onehot_b, (((1,), (0,)), ((), ())),
        preferred_element_type=jnp.float32)
    onef = onehot_b.astype(jnp.float32)
    rank = jnp.sum(onef * (cum - 1.0), axis=1, keepdims=True)
    lt = (idx < lane).astype(jnp.bfloat16)
    ones_row = jnp.ones((1, _B), jnp.bfloat16)
    off = jax.lax.dot_general(
        ones_row, lt, (((1,), (0,)), ((), ())),
        preferred_element_type=jnp.float32)
    pos = rank + jnp.sum(onef * off, axis=1, keepdims=True)
    posi = pos.astype(jnp.int32)
    pos_ref[...] = posi
    off_ref[...] = off.astype(jnp.int32)
    inv = (posi == c).astype(jnp.bfloat16)
    chi = (c // 256).astype(jnp.bfloat16)
    clo = (c % 256).astype(jnp.bfloat16)
    hl = jnp.concatenate([chi, clo], axis=0)
    parts = jax.lax.dot_general(
        hl, inv, (((1,), (0,)), ((), ())),
        preferred_element_type=jnp.float32)
    perm_ref[...] = (256.0 * parts[0:1, :] + parts[1:2, :]).astype(jnp.int32)


def _route(indices):
    return pl.pallas_call(
        _route_body,
        in_specs=[pl.BlockSpec((_B, 1), lambda: (0, 0))],
        out_specs=[
            pl.BlockSpec((_B, 1), lambda: (0, 0)),
            pl.BlockSpec((1, _B), lambda: (0, 0)),
            pl.BlockSpec((1, _EL), lambda: (0, 0)),
        ],
        out_shape=[
            jax.ShapeDtypeStruct((_B, 1), jnp.int32),
            jax.ShapeDtypeStruct((1, _B), jnp.int32),
            jax.ShapeDtypeStruct((1, _EL), jnp.int32),
        ],
    )(indices)


def _sc_mesh():
    return plsc.VectorSubcoreMesh(core_axis_name="core",
                                  subcore_axis_name="subcore")


def _sc_gather_rows(y4, pos4):
    @pl.kernel(out_type=jax.ShapeDtypeStruct((_SB, _SD), y4.dtype),
               mesh=_sc_mesh())
    def k(y_hbm, i_hbm, o_hbm):
        def body(i_vmem, o_vmem):
            pltpu.sync_copy(y_hbm.at[i_vmem.at[0]], o_vmem)

        pltpu.emit_pipeline(
            body,
            grid=(_SB // _SW,),
            in_specs=[pl.BlockSpec((1, _SW), lambda i: (0, i))],
            out_specs=[pl.BlockSpec((_SW, _SD), lambda i: (i, 0))],
            core_axis_name=("core", "subcore"),
            dimension_semantics=(pltpu.PARALLEL,),
        )(i_hbm, o_hbm)

    return k(y4, pos4)


def _mm_body(off_ref, xs_ref, w_ref, y_ref, wbf_ref):
    i = pl.program_id(0)
    t0 = i * _TM

    @pl.when(i == 0)
    def _():
        for e in range(_E):
            wbf_ref[e] = w_ref[e].astype(jnp.bfloat16)

    x = xs_ref[...].astype(jnp.bfloat16)
    riota = jax.lax.broadcasted_iota(jnp.int32, (_TM, 1), 0) + t0
    y_ref[...] = jnp.zeros((_TM, _O), jnp.float32)
    for e in range(_E):
        lo = jnp.maximum(off_ref[e], t0)
        hi = jnp.minimum(off_ref[e + 1], t0 + _TM)

        @pl.when(hi > lo)
        def _(e=e, lo=lo, hi=hi):
            mask = (riota >= lo) & (riota < hi)
            xm = jnp.where(mask, x, jnp.bfloat16(0.0))
            y_ref[...] += jax.lax.dot_general(
                xm, wbf_ref[e], (((1,), (1,)), ((), ())),
                preferred_element_type=jnp.float32)


def _grouped_matmul(off_lanes, x_sorted, W):
    grid_spec = pltpu.PrefetchScalarGridSpec(
        num_scalar_prefetch=1,
        grid=(_B // _TM,),
        in_specs=[
            pl.BlockSpec((_TM, _D), lambda i, off: (i, 0)),
            pl.BlockSpec((_E, _O, _D), lambda i, off: (0, 0, 0)),
        ],
        out_specs=pl.BlockSpec((_TM, _O), lambda i, off: (i, 0)),
        scratch_shapes=[pltpu.VMEM((_E, _O, _D), jnp.bfloat16)],
    )
    return pl.pallas_call(
        _mm_body,
        grid_spec=grid_spec,
        out_shape=jax.ShapeDtypeStruct((_B, _O), jnp.float32),
        compiler_params=pltpu.CompilerParams(
            dimension_semantics=("arbitrary",),
        ),
    )(off_lanes, x_sorted, W)


def _bc_body(y_ref, x_ref, a_ref, lb_ref, out_ref):
    y = y_ref[...]
    x = x_ref[...]
    a2 = _SCALE * a_ref[...] * jnp.sum(lb_ref[...], axis=1, keepdims=True)
    sz = jax.lax.dot_general(
        x, a2, (((1,), (1,)), ((), ())),
        preferred_element_type=jnp.float32)
    for e in range(_E):
        out_ref[:, e, :] = y + sz[:, e:e + 1]


def _broadcast_lora(y_tok, x, a_mat, lb_mat):
    return pl.pallas_call(
        _bc_body,
        grid=(_B // _TM,),
        in_specs=[
            pl.BlockSpec((_TM, _O), lambda i: (i, 0)),
            pl.BlockSpec((_TM, _D), lambda i: (i, 0)),
            pl.BlockSpec((_E, _D), lambda i: (0, 0)),
            pl.BlockSpec((_E, _O), lambda i: (0, 0)),
        ],
        out_specs=pl.BlockSpec((_TM, _E, _O), lambda i: (i, 0, 0)),
        out_shape=jax.ShapeDtypeStruct((_B, _E, _O), jnp.float32),
        compiler_params=pltpu.CompilerParams(
            dimension_semantics=("arbitrary",),
        ),
    )(y_tok, x, a_mat, lb_mat)


def kernel(x, indices, W, lora_a, lora_b):
    pos, perm, off = _route(indices)
    arange4 = jnp.arange(_SPLIT, dtype=jnp.int32)
    pos4 = (_SPLIT * pos + arange4[None, :]).reshape(1, _SB)
    perm4 = (_SPLIT * perm.reshape(_B, 1) + arange4[None, :]).reshape(1, _SB)
    off_lanes = off.reshape(_EL)
    x_sorted = _sc_gather_rows(x.reshape(_SB, _SD), perm4).reshape(_B, _D)
    y_sorted = _grouped_matmul(off_lanes, x_sorted, W)
    y_tok = _sc_gather_rows(y_sorted.reshape(_SB, _SD), pos4).reshape(_B, _O)
    return _broadcast_lora(y_tok, x, lora_a.reshape(_E, _D),
                           lora_b.reshape(_E, _O))


# speedup vs baseline: 1.3498x; 1.3498x over previous
import jax
import jax.numpy as jnp
from jax.experimental import pallas as pl
from jax.experimental.pallas import tpu as pltpu
from jax.experimental.pallas import tpu_sc as plsc

_E = 8
_EL = 16
_D = 1024
_O = 1024
_B = 2048
_SCALE = 20.0
_KP = 4
_PD = _D // _KP
_PB = _B * _KP
_TMM = 256
_TBC = 128
_SW = 128


def _route_body(idx_ref, x_ref, a_ref, lb_ref,
                xp_ref, pos_ref, perm_ref, off_ref, sz_ref):
    k = pl.program_id(0)
    xcol = x_ref[...]
    xp_ref[0] = xcol
    colsum = jnp.sum(lb_ref[...], axis=1, keepdims=True)
    a2 = _SCALE * a_ref[...] * colsum
    part = jax.lax.dot_general(
        xcol, a2, (((1,), (1,)), ((), ())),
        preferred_element_type=jnp.float32)

    @pl.when(k == 0)
    def _():
        sz_ref[...] = jnp.zeros((_B, _E), jnp.float32)
        idx = idx_ref[...]
        lane = jax.lax.broadcasted_iota(jnp.int32, (1, _EL), 1)
        onehot_b = (idx == lane).astype(jnp.bfloat16)
        r = jax.lax.broadcasted_iota(jnp.int32, (_B, 1), 0)
        c = jax.lax.broadcasted_iota(jnp.int32, (1, _B), 1)
        tril = (r >= c).astype(jnp.bfloat16)
        cum = jax.lax.dot_general(
            tril, onehot_b, (((1,), (0,)), ((), ())),
            preferred_element_type=jnp.float32)
        onef = onehot_b.astype(jnp.float32)
        rank = jnp.sum(onef * (cum - 1.0), axis=1, keepdims=True)
        lt = (idx < lane).astype(jnp.bfloat16)
        ones_row = jnp.ones((1, _B), jnp.bfloat16)
        off = jax.lax.dot_general(
            ones_row, lt, (((1,), (0,)), ((), ())),
            preferred_element_type=jnp.float32)
        pos = rank + jnp.sum(onef * off, axis=1, keepdims=True)
        posi = pos.astype(jnp.int32)
        pos_ref[...] = posi
        off_ref[...] = off.astype(jnp.int32)
        inv = (posi == c).astype(jnp.bfloat16)
        chi = (c // 256).astype(jnp.bfloat16)
        clo = (c % 256).astype(jnp.bfloat16)
        hl = jnp.concatenate([chi, clo], axis=0)
        parts = jax.lax.dot_general(
            hl, inv, (((1,), (0,)), ((), ())),
            preferred_element_type=jnp.float32)
        perm_ref[...] = (256.0 * parts[0:1, :]
                         + parts[1:2, :]).astype(jnp.int32)

    sz_ref[...] += part


def _route(indices, x, a_mat, lb_mat):
    return pl.pallas_call(
        _route_body,
        grid=(_KP,),
        in_specs=[
            pl.BlockSpec((_B, 1), lambda k: (0, 0)),
            pl.BlockSpec((_B, _PD), lambda k: (0, k)),
            pl.BlockSpec((_E, _PD), lambda k: (0, k)),
            pl.BlockSpec((_E, _O), lambda k: (0, 0)),
        ],
        out_specs=[
            pl.BlockSpec((1, _B, _PD), lambda k: (k, 0, 0)),
            pl.BlockSpec((_B, 1), lambda k: (0, 0)),
            pl.BlockSpec((1, _B), lambda k: (0, 0)),
            pl.BlockSpec((1, _EL), lambda k: (0, 0)),
            pl.BlockSpec((_B, _E), lambda k: (0, 0)),
        ],
        out_shape=[
            jax.ShapeDtypeStruct((_KP, _B, _PD), jnp.float32),
            jax.ShapeDtypeStruct((_B, 1), jnp.int32),
            jax.ShapeDtypeStruct((1, _B), jnp.int32),
            jax.ShapeDtypeStruct((1, _EL), jnp.int32),
            jax.ShapeDtypeStruct((_B, _E), jnp.float32),
        ],
        compiler_params=pltpu.CompilerParams(
            dimension_semantics=("arbitrary",),
        ),
    )(indices, x, a_mat, lb_mat)


def _sc_mesh():
    return plsc.VectorSubcoreMesh(core_axis_name="core",
                                  subcore_axis_name="subcore")


def _sc_gather_rows(y, idx_row):
    @pl.kernel(out_type=jax.ShapeDtypeStruct((_PB, _PD), y.dtype),
               mesh=_sc_mesh())
    def k(y_hbm, i_hbm, o_hbm):
        def body(i_vmem, o_vmem):
            pltpu.sync_copy(y_hbm.at[i_vmem.at[0]], o_vmem)

        pltpu.emit_pipeline(
            body,
            grid=(_PB // _SW,),
            in_specs=[pl.BlockSpec((1, _SW), lambda i: (0, i))],
            out_specs=[pl.BlockSpec((_SW, _PD), lambda i: (i, 0))],
            core_axis_name=("core", "subcore"),
            dimension_semantics=(pltpu.PARALLEL,),
        )(i_hbm, o_hbm)

    return k(y, idx_row)


def _mm_body(off_ref, xp_ref, w_ref, yp_ref, wbf_ref, acc_ref):
    i = pl.program_id(0)
    t0 = i * _TMM

    @pl.when(i == 0)
    def _():
        for e in range(_E):
            wbf_ref[e] = w_ref[e].astype(jnp.bfloat16)

    xk = [xp_ref[k].astype(jnp.bfloat16) for k in range(_KP)]
    riota = jax.lax.broadcasted_iota(jnp.int32, (_TMM, 1), 0) + t0
    acc_ref[...] = jnp.zeros((_TMM, _O), jnp.float32)
    for e in range(_E):
        lo = jnp.maximum(off_ref[e], t0)
        hi = jnp.minimum(off_ref[e + 1], t0 + _TMM)

        @pl.when(hi > lo)
        def _(e=e, lo=lo, hi=hi):
            mask = (riota >= lo) & (riota < hi)
            for k in range(_KP):
                xm = jnp.where(mask, xk[k], jnp.bfloat16(0.0))
                acc_ref[...] += jax.lax.dot_general(
                    xm, wbf_ref[e][:, k * _PD:(k + 1) * _PD],
                    (((1,), (1,)), ((), ())),
                    preferred_element_type=jnp.float32)

    acc = acc_ref[...]
    for k in range(_KP):
        yp_ref[k] = acc[:, k * _PD:(k + 1) * _PD]


def _grouped_matmul(off_lanes, xp_sorted, W):
    nt = _B // _TMM
    grid_spec = pltpu.PrefetchScalarGridSpec(
        num_scalar_prefetch=1,
        grid=(nt,),
        in_specs=[
            pl.BlockSpec((_KP, _TMM, _PD), lambda i, off: (0, i, 0)),
            pl.BlockSpec((_E, _O, _D), lambda i, off: (0, 0, 0)),
        ],
        out_specs=pl.BlockSpec((_KP, _TMM, _PD), lambda i, off: (0, i, 0)),
        scratch_shapes=[
            pltpu.VMEM((_E, _O, _D), jnp.bfloat16),
            pltpu.VMEM((_TMM, _O), jnp.float32),
        ],
    )
    return pl.pallas_call(
        _mm_body,
        grid_spec=grid_spec,
        out_shape=jax.ShapeDtypeStruct((_KP, _B, _PD), jnp.float32),
        compiler_params=pltpu.CompilerParams(
            dimension_semantics=("arbitrary",),
        ),
    )(off_lanes, xp_sorted, W)


def _bc_body(yp_ref, sz_ref, out_ref):
    y = jnp.concatenate([yp_ref[k] for k in range(_KP)], axis=1)
    sz = sz_ref[...]
    for e in range(_E):
        out_ref[:, e, :] = y + sz[:, e:e + 1]


def _broadcast_lora(yp_tok, sz):
    nt = _B // _TBC
    return pl.pallas_call(
        _bc_body,
        grid=(nt,),
        in_specs=[
            pl.BlockSpec((_KP, _TBC, _PD), lambda i: (0, i, 0)),
            pl.BlockSpec((_TBC, _E), lambda i: (i, 0)),
        ],
        out_specs=pl.BlockSpec((_TBC, _E, _O), lambda i: (i, 0, 0)),
        out_shape=jax.ShapeDtypeStruct((_B, _E, _O), jnp.float32),
        compiler_params=pltpu.CompilerParams(
            dimension_semantics=("arbitrary",),
        ),
    )(yp_tok, sz)


def kernel(x, indices, W, lora_a, lora_b):
    xp, pos, perm, off, sz = _route(indices, x, lora_a.reshape(_E, _D),
                                    lora_b.reshape(_E, _O))
    off_lanes = off.reshape(_EL)
    plane_base = _B * jnp.arange(_KP, dtype=jnp.int32)[:, None]
    perm_pl = (perm + plane_base).reshape(1, _PB)
    pos_pl = (pos.reshape(1, _B) + plane_base).reshape(1, _PB)
    xp_sorted = _sc_gather_rows(xp.reshape(_PB, _PD), perm_pl)
    yp_sorted = _grouped_matmul(off_lanes, xp_sorted.reshape(_KP, _B, _PD), W)
    yp_tok = _sc_gather_rows(yp_sorted.reshape(_PB, _PD), pos_pl)
    return _broadcast_lora(yp_tok.reshape(_KP, _B, _PD), sz)


# speedup vs baseline: 1.3844x; 1.0256x over previous
import jax
import jax.numpy as jnp
from jax.experimental import pallas as pl
from jax.experimental.pallas import tpu as pltpu
from jax.experimental.pallas import tpu_sc as plsc

_E = 8
_EL = 16
_D = 1024
_O = 1024
_B = 2048
_SCALE = 20.0
_KP = 4
_PD = _D // _KP
_PB = _B * _KP
_TMM = 256
_TBC = 256
_SW = 128


def _route_body(idx_ref, x_ref, a_ref, lb_ref,
                xp_ref, pos_ref, perm_ref, off_ref, sz_ref):
    k = pl.program_id(0)
    xcol = x_ref[...]
    xp_ref[0] = xcol
    colsum = jnp.sum(lb_ref[...], axis=1, keepdims=True)
    a2 = _SCALE * a_ref[...] * colsum
    part = jax.lax.dot_general(
        xcol, a2, (((1,), (1,)), ((), ())),
        preferred_element_type=jnp.float32)

    @pl.when(k == 0)
    def _():
        sz_ref[...] = jnp.zeros((_B, _E), jnp.float32)
        idx = idx_ref[...]
        lane = jax.lax.broadcasted_iota(jnp.int32, (1, _EL), 1)
        onehot_b = (idx == lane).astype(jnp.bfloat16)
        r = jax.lax.broadcasted_iota(jnp.int32, (_B, 1), 0)
        c = jax.lax.broadcasted_iota(jnp.int32, (1, _B), 1)
        tril = (r >= c).astype(jnp.bfloat16)
        cum = jax.lax.dot_general(
            tril, onehot_b, (((1,), (0,)), ((), ())),
            preferred_element_type=jnp.float32)
        onef = onehot_b.astype(jnp.float32)
        rank = jnp.sum(onef * (cum - 1.0), axis=1, keepdims=True)
        lt = (idx < lane).astype(jnp.bfloat16)
        ones_row = jnp.ones((1, _B), jnp.bfloat16)
        off = jax.lax.dot_general(
            ones_row, lt, (((1,), (0,)), ((), ())),
            preferred_element_type=jnp.float32)
        pos = rank + jnp.sum(onef * off, axis=1, keepdims=True)
        posi = pos.astype(jnp.int32)
        pos_ref[...] = posi
        off_ref[...] = off.astype(jnp.int32)
        inv = (posi == c).astype(jnp.bfloat16)
        chi = (c // 256).astype(jnp.bfloat16)
        clo = (c % 256).astype(jnp.bfloat16)
        hl = jnp.concatenate([chi, clo], axis=0)
        parts = jax.lax.dot_general(
            hl, inv, (((1,), (0,)), ((), ())),
            preferred_element_type=jnp.float32)
        perm_ref[...] = (256.0 * parts[0:1, :]
                         + parts[1:2, :]).astype(jnp.int32)

    sz_ref[...] += part


def _route(indices, x, a_mat, lb_mat):
    return pl.pallas_call(
        _route_body,
        grid=(_KP,),
        in_specs=[
            pl.BlockSpec((_B, 1), lambda k: (0, 0)),
            pl.BlockSpec((_B, _PD), lambda k: (0, k)),
            pl.BlockSpec((_E, _PD), lambda k: (0, k)),
            pl.BlockSpec((_E, _O), lambda k: (0, 0)),
        ],
        out_specs=[
            pl.BlockSpec((1, _B, _PD), lambda k: (k, 0, 0)),
            pl.BlockSpec((_B, 1), lambda k: (0, 0)),
            pl.BlockSpec((1, _B), lambda k: (0, 0)),
            pl.BlockSpec((1, _EL), lambda k: (0, 0)),
            pl.BlockSpec((_B, _E), lambda k: (0, 0)),
        ],
        out_shape=[
            jax.ShapeDtypeStruct((_KP, _B, _PD), jnp.float32),
            jax.ShapeDtypeStruct((_B, 1), jnp.int32),
            jax.ShapeDtypeStruct((1, _B), jnp.int32),
            jax.ShapeDtypeStruct((1, _EL), jnp.int32),
            jax.ShapeDtypeStruct((_B, _E), jnp.float32),
        ],
        compiler_params=pltpu.CompilerParams(
            dimension_semantics=("arbitrary",),
        ),
    )(indices, x, a_mat, lb_mat)


def _sc_mesh():
    return plsc.VectorSubcoreMesh(core_axis_name="core",
                                  subcore_axis_name="subcore")


def _sc_gather_rows(y, idx_row):
    @pl.kernel(out_type=jax.ShapeDtypeStruct((_PB, _PD), y.dtype),
               mesh=_sc_mesh())
    def k(y_hbm, i_hbm, o_hbm):
        def body(i_vmem, o_vmem):
            pltpu.sync_copy(y_hbm.at[i_vmem.at[0]], o_vmem)

        pltpu.emit_pipeline(
            body,
            grid=(_PB // _SW,),
            in_specs=[pl.BlockSpec((1, _SW), lambda i: (0, i))],
            out_specs=[pl.BlockSpec((_SW, _PD), lambda i: (i, 0))],
            core_axis_name=("core", "subcore"),
            dimension_semantics=(pltpu.PARALLEL,),
        )(i_hbm, o_hbm)

    return k(y, idx_row)


def _mm_body(off_ref, xp_ref, w_ref, yp_ref, wbf_ref, acc_ref):
    i = pl.program_id(0)
    t0 = i * _TMM

    @pl.when(i == 0)
    def _():
        for e in range(_E):
            wbf_ref[e] = w_ref[e].astype(jnp.bfloat16)

    xk = [xp_ref[k].astype(jnp.bfloat16) for k in range(_KP)]
    riota = jax.lax.broadcasted_iota(jnp.int32, (_TMM, 1), 0) + t0
    acc_ref[...] = jnp.zeros((_TMM, _O), jnp.float32)
    for e in range(_E):
        lo = jnp.maximum(off_ref[e], t0)
        hi = jnp.minimum(off_ref[e + 1], t0 + _TMM)

        @pl.when(hi > lo)
        def _(e=e, lo=lo, hi=hi):
            mask = (riota >= lo) & (riota < hi)
            for k in range(_KP):
                xm = jnp.where(mask, xk[k], jnp.bfloat16(0.0))
                acc_ref[...] += jax.lax.dot_general(
                    xm, wbf_ref[e][:, k * _PD:(k + 1) * _PD],
                    (((1,), (1,)), ((), ())),
                    preferred_element_type=jnp.float32)

    acc = acc_ref[...]
    for k in range(_KP):
        yp_ref[k] = acc[:, k * _PD:(k + 1) * _PD]


def _grouped_matmul(off_lanes, xp_sorted, W):
    nt = _B // _TMM
    grid_spec = pltpu.PrefetchScalarGridSpec(
        num_scalar_prefetch=1,
        grid=(nt,),
        in_specs=[
            pl.BlockSpec((_KP, _TMM, _PD), lambda i, off: (0, i, 0)),
            pl.BlockSpec((_E, _O, _D), lambda i, off: (0, 0, 0)),
        ],
        out_specs=pl.BlockSpec((_KP, _TMM, _PD), lambda i, off: (0, i, 0)),
        scratch_shapes=[
            pltpu.VMEM((_E, _O, _D), jnp.bfloat16),
            pltpu.VMEM((_TMM, _O), jnp.float32),
        ],
    )
    return pl.pallas_call(
        _mm_body,
        grid_spec=grid_spec,
        out_shape=jax.ShapeDtypeStruct((_KP, _B, _PD), jnp.float32),
        compiler_params=pltpu.CompilerParams(
            dimension_semantics=("arbitrary",),
        ),
    )(off_lanes, xp_sorted, W)


def _bc_body(yp_ref, sz_ref, out_ref):
    y = jnp.concatenate([yp_ref[k] for k in range(_KP)], axis=1)
    sz = sz_ref[...]
    for e in range(_E):
        out_ref[:, e, :] = y + sz[:, e:e + 1]


def _broadcast_lora(yp_tok, sz):
    nt = _B // _TBC
    return pl.pallas_call(
        _bc_body,
        grid=(nt,),
        in_specs=[
            pl.BlockSpec((_KP, _TBC, _PD), lambda i: (0, i, 0)),
            pl.BlockSpec((_TBC, _E), lambda i: (i, 0)),
        ],
        out_specs=pl.BlockSpec((_TBC, _E, _O), lambda i: (i, 0, 0)),
        out_shape=jax.ShapeDtypeStruct((_B, _E, _O), jnp.float32),
        compiler_params=pltpu.CompilerParams(
            dimension_semantics=("arbitrary",),
        ),
    )(yp_tok, sz)


def kernel(x, indices, W, lora_a, lora_b):
    xp, pos, perm, off, sz = _route(indices, x, lora_a.reshape(_E, _D),
                                    lora_b.reshape(_E, _O))
    off_lanes = off.reshape(_EL)
    plane_base = _B * jnp.arange(_KP, dtype=jnp.int32)[:, None]
    perm_pl = (perm + plane_base).reshape(1, _PB)
    pos_pl = (pos.reshape(1, _B) + plane_base).reshape(1, _PB)
    xp_sorted = _sc_gather_rows(xp.reshape(_PB, _PD), perm_pl)
    yp_sorted = _grouped_matmul(off_lanes, xp_sorted.reshape(_KP, _B, _PD), W)
    yp_tok = _sc_gather_rows(yp_sorted.reshape(_PB, _PD), pos_pl)
    return _broadcast_lora(yp_tok.reshape(_KP, _B, _PD), sz)
